# baseline (speedup 1.0000x reference)
import jax
import jax.numpy as jnp
from jax.experimental import pallas as pl
from jax.experimental.pallas import tpu as pltpu

_CONF = 0.5
_THR = 50.0 / 800.0
_MAXL = 5
_NSTRIPS = 71.0
_NCOLS = 76
_N = 20000
_DEAD = -1e9


def _nms_kernel(pt_ref, pred_ref, kept_ref, keep_ref, num_ref,
                state_ref, acc_ref, row_ref, sem):
    p0 = pt_ref[0:1, :]
    p1 = pt_ref[1:2, :]
    m = jnp.maximum(p0, p1)
    e0 = jnp.exp(p0 - m)
    e1 = jnp.exp(p1 - m)
    score = e1 / (e0 + e1)
    state_ref[0:1, :] = jnp.where(score >= _CONF, score, _DEAD)
    p2 = pt_ref[2:3, :]
    p3 = pt_ref[3:4, :]
    state_ref[1:2, :] = jnp.clip(jnp.round(p2 * _NSTRIPS), 0.0, _NSTRIPS)
    state_ref[2:3, :] = jnp.clip(
        jnp.round(p2 * _NSTRIPS + p3 * _NSTRIPS - 1.0), 0.0, _NSTRIPS)
    num_ref[0] = 0

    lane76 = jax.lax.broadcasted_iota(jnp.int32, (1, _NCOLS), 1)
    srow = jax.lax.broadcasted_iota(jnp.int32, (8, 1), 0)

    def it(t, carry):
        live = state_ref[0:1, :]
        lidx = jax.lax.broadcasted_iota(jnp.int32, (1, _N), 1)
        mx = jnp.max(live)
        bi = jnp.min(jnp.where(live == mx, lidx, _N))
        valid = mx > -1e8
        sel = (lidx == bi).astype(jnp.float32)

        cp = pltpu.make_async_copy(
            pred_ref.at[pl.ds(bi, 1), :], row_ref, sem)
        cp.start()
        cp.wait()
        row = row_ref[...]
        rb2 = row[:, 2:3]
        rb3 = row[:, 3:4]
        sb = jnp.clip(jnp.round(rb2 * _NSTRIPS), 0.0, _NSTRIPS)
        eb = jnp.clip(jnp.round(rb2 * _NSTRIPS + rb3 * _NSTRIPS - 1.0),
                      0.0, _NSTRIPS)

        start = state_ref[1:2, :]
        end = state_ref[2:3, :]
        s = jnp.maximum(start, sb)
        e = jnp.minimum(end, eb)
        cnt = jnp.maximum(e - s + 1.0, 0.0)

        rs = jax.lax.div(jnp.max(sb).astype(jnp.int32), 8)
        re = jax.lax.div(jnp.max(eb).astype(jnp.int32), 8)
        acc_ref[...] = jnp.zeros((8, _N), jnp.float32)

        def rowblk(rf, c):
            base = pl.multiple_of(8 + 8 * rf, 8)
            xsr = pt_ref[pl.ds(base, 8), :]
            xbr = jnp.sum(xsr * sel, axis=1, keepdims=True)
            kr = (srow + 8 * rf).astype(jnp.float32)
            maskf = ((kr >= s) & (kr <= e)).astype(jnp.float32)
            acc_ref[...] = acc_ref[...] + jnp.abs(xsr - xbr) * maskf
            return c

        jax.lax.fori_loop(rs, re + 1, rowblk, 0)
        dist = jnp.sum(acc_ref[...], axis=0, keepdims=True)

        supp = ((dist < cnt * _THR) & (cnt > 0.0)) | (lidx == bi)
        state_ref[0:1, :] = jnp.where(supp & valid, _DEAD, live)

        validf = valid.astype(jnp.float32)
        col3 = jnp.round(rb3 * _NSTRIPS) * validf
        out_row = jnp.where(lane76 == 3, col3, row * validf)
        kept_ref[pl.ds(t, 1), :] = out_row
        keep_ref[t] = jnp.where(valid, bi, jnp.int32(-1))
        num_ref[0] = num_ref[0] + valid.astype(jnp.int32)
        return carry

    jax.lax.fori_loop(0, _MAXL, it, 0)


def kernel(predictions):
    pt = predictions.T
    ptp = jnp.concatenate(
        [pt[:4], jnp.zeros((4, _N), jnp.float32), pt[4:]], axis=0)
    kept, keep, num = pl.pallas_call(
        _nms_kernel,
        out_shape=(
            jax.ShapeDtypeStruct((_MAXL, _NCOLS), jnp.float32),
            jax.ShapeDtypeStruct((_MAXL,), jnp.int32),
            jax.ShapeDtypeStruct((1,), jnp.int32),
        ),
        in_specs=[
            pl.BlockSpec(memory_space=pltpu.VMEM),
            pl.BlockSpec(memory_space=pltpu.MemorySpace.HBM),
        ],
        out_specs=(
            pl.BlockSpec(memory_space=pltpu.VMEM),
            pl.BlockSpec(memory_space=pltpu.SMEM),
            pl.BlockSpec(memory_space=pltpu.SMEM),
        ),
        scratch_shapes=[
            pltpu.VMEM((8, _N), jnp.float32),
            pltpu.VMEM((8, _N), jnp.float32),
            pltpu.VMEM((1, _NCOLS), jnp.float32),
            pltpu.SemaphoreType.DMA,
        ],
    )(ptp, predictions)
    return kept, keep, num[0]


# speedup vs baseline: 1.0477x; 1.0477x over previous
import jax
import jax.numpy as jnp
from jax.experimental import pallas as pl
from jax.experimental.pallas import tpu as pltpu

_CONF = 0.5
_THR = 50.0 / 800.0
_MAXL = 5
_NSTRIPS = 71.0
_NCOLS = 76
_N = 20000
_DEAD = -1e9


def _nms_kernel(pt_ref, pred_ref, kept_ref, keep_ref, num_ref,
                state_ref, row_ref, sem):
    p0 = pt_ref[0:1, :]
    p1 = pt_ref[1:2, :]
    m = jnp.maximum(p0, p1)
    e0 = jnp.exp(p0 - m)
    e1 = jnp.exp(p1 - m)
    score = e1 / (e0 + e1)
    state_ref[0:1, :] = jnp.where(score >= _CONF, score, _DEAD)
    p2 = pt_ref[2:3, :]
    p3 = pt_ref[3:4, :]
    state_ref[1:2, :] = jnp.clip(jnp.round(p2 * _NSTRIPS), 0.0, _NSTRIPS)
    state_ref[2:3, :] = jnp.clip(
        jnp.round(p2 * _NSTRIPS + p3 * _NSTRIPS - 1.0), 0.0, _NSTRIPS)
    num_ref[0] = 0

    lane76 = jax.lax.broadcasted_iota(jnp.int32, (1, _NCOLS), 1)
    srow = jax.lax.broadcasted_iota(jnp.int32, (8, 1), 0)

    def it(t, carry):
        live = state_ref[0:1, :]
        lidx = jax.lax.broadcasted_iota(jnp.int32, (1, _N), 1)
        mx = jnp.max(live)
        bi = jnp.min(jnp.where(live == mx, lidx, _N))
        valid = mx > -1e8
        sel = (lidx == bi).astype(jnp.float32)

        cp = pltpu.make_async_copy(
            pred_ref.at[pl.ds(bi, 1), :], row_ref, sem)
        cp.start()
        cp.wait()
        row = row_ref[...]
        rb2 = row[:, 2:3]
        rb3 = row[:, 3:4]
        sb = jnp.clip(jnp.round(rb2 * _NSTRIPS), 0.0, _NSTRIPS)
        eb = jnp.clip(jnp.round(rb2 * _NSTRIPS + rb3 * _NSTRIPS - 1.0),
                      0.0, _NSTRIPS)

        start = state_ref[1:2, :]
        end = state_ref[2:3, :]
        s = jnp.maximum(start, sb)
        e = jnp.minimum(end, eb)
        cnt = jnp.maximum(e - s + 1.0, 0.0)

        acc = jnp.zeros((8, _N), jnp.float32)
        for r in range(9):
            xsr = pt_ref[pl.ds(8 + 8 * r, 8), :]
            xbr = jnp.sum(xsr * sel, axis=1, keepdims=True)
            kr = (srow + 8 * r).astype(jnp.float32)
            maskf = ((kr >= s) & (kr <= e)).astype(jnp.float32)
            acc = acc + jnp.abs(xsr - xbr) * maskf
        dist = jnp.sum(acc, axis=0, keepdims=True)

        supp = ((dist < cnt * _THR) & (cnt > 0.0)) | (lidx == bi)
        state_ref[0:1, :] = jnp.where(supp & valid, _DEAD, live)

        validf = valid.astype(jnp.float32)
        col3 = jnp.round(rb3 * _NSTRIPS) * validf
        out_row = jnp.where(lane76 == 3, col3, row * validf)
        kept_ref[pl.ds(t, 1), :] = out_row
        keep_ref[t] = jnp.where(valid, bi, jnp.int32(-1))
        num_ref[0] = num_ref[0] + valid.astype(jnp.int32)
        return carry

    jax.lax.fori_loop(0, _MAXL, it, 0)


def kernel(predictions):
    pt = predictions.T
    ptp = jnp.concatenate(
        [pt[:4], jnp.zeros((4, _N), jnp.float32), pt[4:]], axis=0)
    kept, keep, num = pl.pallas_call(
        _nms_kernel,
        out_shape=(
            jax.ShapeDtypeStruct((_MAXL, _NCOLS), jnp.float32),
            jax.ShapeDtypeStruct((_MAXL,), jnp.int32),
            jax.ShapeDtypeStruct((1,), jnp.int32),
        ),
        in_specs=[
            pl.BlockSpec(memory_space=pltpu.VMEM),
            pl.BlockSpec(memory_space=pltpu.MemorySpace.HBM),
        ],
        out_specs=(
            pl.BlockSpec(memory_space=pltpu.VMEM),
            pl.BlockSpec(memory_space=pltpu.SMEM),
            pl.BlockSpec(memory_space=pltpu.SMEM),
        ),
        scratch_shapes=[
            pltpu.VMEM((8, _N), jnp.float32),
            pltpu.VMEM((1, _NCOLS), jnp.float32),
            pltpu.SemaphoreType.DMA,
        ],
    )(ptp, predictions)
    return kept, keep, num[0]


# speedup vs baseline: 1.0880x; 1.0384x over previous
import jax
import jax.numpy as jnp
from jax.experimental import pallas as pl
from jax.experimental.pallas import tpu as pltpu

_CONF = 0.5
_THR = 50.0 / 800.0
_MAXL = 5
_NSTRIPS = 71.0
_NCOLS = 76
_N = 20000
_DEAD = -1e9


def _nms_kernel(pt_ref, pred_ref, kept_ref, keep_ref, num_ref,
                state_ref, row_ref, xcol_ref, sem):
    p0 = pt_ref[0:1, :]
    p1 = pt_ref[1:2, :]
    m = jnp.maximum(p0, p1)
    e0 = jnp.exp(p0 - m)
    e1 = jnp.exp(p1 - m)
    score = e1 / (e0 + e1)
    state_ref[0:1, :] = jnp.where(score >= _CONF, score, _DEAD)
    p2 = pt_ref[2:3, :]
    p3 = pt_ref[3:4, :]
    state_ref[1:2, :] = jnp.clip(jnp.round(p2 * _NSTRIPS), 0.0, _NSTRIPS)
    state_ref[2:3, :] = jnp.clip(
        jnp.round(p2 * _NSTRIPS + p3 * _NSTRIPS - 1.0), 0.0, _NSTRIPS)
    num_ref[0] = 0

    lane76 = jax.lax.broadcasted_iota(jnp.int32, (1, _NCOLS), 1)
    srow = jax.lax.broadcasted_iota(jnp.int32, (8, 1), 0)

    def it(t, carry):
        live = state_ref[0:1, :]
        lidx = jax.lax.broadcasted_iota(jnp.int32, (1, _N), 1)
        mx = jnp.max(live)
        bi = jnp.min(jnp.where(live == mx, lidx, _N))
        valid = mx > -1e8
        sel = (lidx == bi).astype(jnp.float32)

        cp = pltpu.make_async_copy(
            pred_ref.at[pl.ds(bi, 1), :], row_ref, sem)
        cp.start()
        cp.wait()
        row = row_ref[...]
        rb2 = row[:, 2:3]
        rb3 = row[:, 3:4]
        sb = jnp.clip(jnp.round(rb2 * _NSTRIPS), 0.0, _NSTRIPS)
        eb = jnp.clip(jnp.round(rb2 * _NSTRIPS + rb3 * _NSTRIPS - 1.0),
                      0.0, _NSTRIPS)

        start = state_ref[1:2, :]
        end = state_ref[2:3, :]
        s = jnp.maximum(start, sb)
        e = jnp.minimum(end, eb)
        cnt = jnp.maximum(e - s + 1.0, 0.0)

        xcol = jnp.transpose(row[:, 4:], (1, 0))
        acc = jnp.zeros((8, _N), jnp.float32)
        for r in range(9):
            xsr = pt_ref[pl.ds(8 + 8 * r, 8), :]
            xbr = xcol[8 * r:8 * r + 8, :]
            kr = (srow + 8 * r).astype(jnp.float32)
            maskf = ((kr >= s) & (kr <= e)).astype(jnp.float32)
            acc = acc + jnp.abs(xsr - xbr) * maskf
        dist = jnp.sum(acc, axis=0, keepdims=True)

        supp = ((dist < cnt * _THR) & (cnt > 0.0)) | (lidx == bi)
        state_ref[0:1, :] = jnp.where(supp & valid, _DEAD, live)

        validf = valid.astype(jnp.float32)
        col3 = jnp.round(rb3 * _NSTRIPS) * validf
        out_row = jnp.where(lane76 == 3, col3, row * validf)
        kept_ref[pl.ds(t, 1), :] = out_row
        keep_ref[t] = jnp.where(valid, bi, jnp.int32(-1))
        num_ref[0] = num_ref[0] + valid.astype(jnp.int32)
        return carry

    jax.lax.fori_loop(0, _MAXL, it, 0)


def kernel(predictions):
    pt = predictions.T
    ptp = jnp.concatenate(
        [pt[:4], jnp.zeros((4, _N), jnp.float32), pt[4:]], axis=0)
    kept, keep, num = pl.pallas_call(
        _nms_kernel,
        out_shape=(
            jax.ShapeDtypeStruct((_MAXL, _NCOLS), jnp.float32),
            jax.ShapeDtypeStruct((_MAXL,), jnp.int32),
            jax.ShapeDtypeStruct((1,), jnp.int32),
        ),
        in_specs=[
            pl.BlockSpec(memory_space=pltpu.VMEM),
            pl.BlockSpec(memory_space=pltpu.MemorySpace.HBM),
        ],
        out_specs=(
            pl.BlockSpec(memory_space=pltpu.VMEM),
            pl.BlockSpec(memory_space=pltpu.SMEM),
            pl.BlockSpec(memory_space=pltpu.SMEM),
        ),
        scratch_shapes=[
            pltpu.VMEM((8, _N), jnp.float32),
            pltpu.VMEM((1, _NCOLS), jnp.float32),
            pltpu.VMEM((80, 1), jnp.float32),
            pltpu.SemaphoreType.DMA,
        ],
    )(ptp, predictions)
    return kept, keep, num[0]
